# baseline (speedup 1.0000x reference)
import functools

import jax
import jax.numpy as jnp
from jax.experimental import pallas as pl
from jax.experimental.pallas import tpu as pltpu

_NWIN = 7
_NREG = _NWIN * _NWIN
_NEG = -1e30
_BIGNEG = -1e9


def _ln(x, g, b):
    mu = jnp.mean(x, axis=-1, keepdims=True)
    var = jnp.mean((x - mu) ** 2, axis=-1, keepdims=True)
    return (x - mu) * jax.lax.rsqrt(var + 1e-6) * g + b


def _softmax(s):
    e = jnp.exp(s)
    return e / jnp.sum(e, axis=-1, keepdims=True)


def _topk_mask(aff, topk):
    work = aff
    col = jax.lax.broadcasted_iota(jnp.int32, (_NREG, _NREG), 1)
    mask = jnp.zeros((_NREG, _NREG), jnp.float32)
    pick0 = None
    for t in range(topk):
        cmax = jnp.max(work, axis=-1, keepdims=True)
        pick = jnp.min(jnp.where(work >= cmax, col, _NREG),
                       axis=-1, keepdims=True)
        if t == 0:
            pick0 = pick
        first = col == pick
        mask = mask + first.astype(jnp.float32)
        work = jnp.where(first, _NEG, work)
    return mask, pick0


def _region_matrix(ho, wo, hh, ww):
    y = jax.lax.broadcasted_iota(jnp.int32, (ho, wo, _NREG), 0)
    x = jax.lax.broadcasted_iota(jnp.int32, (ho, wo, _NREG), 1)
    r = jax.lax.broadcasted_iota(jnp.int32, (ho, wo, _NREG), 2)
    e3 = ((y // hh) * _NWIN + (x // ww)) == r
    return e3.astype(jnp.float32).reshape(ho * wo, _NREG)


def _stage_kernel(patches_ref, wd_ref, dsb_ref, dslng_ref, dslnb_ref,
                  ln1g_ref, ln1b_ref, wqkv_ref, wo_ref,
                  ln2g_ref, ln2b_ref, mlp1_ref, mlp1b_ref,
                  mlp2_ref, mlp2b_ref, nhwc_ref, nchw_ref, *scratch,
                  ho, wo, hh, ww, nh, topk, wp):
    C = wo_ref.shape[0]
    dh = C // nh
    hw = hh * ww
    T = _NREG * hw
    f32 = jnp.float32
    bf16 = jnp.bfloat16
    i32 = jnp.int32
    dot = functools.partial(jax.lax.dot_general, preferred_element_type=f32)
    mm = lambda a, b: dot(a, b, (((1,), (0,)), ((), ())))
    mm_nt = lambda a, b: dot(a, b, (((1,), (1,)), ((), ())))
    mm_tn = lambda a, b: dot(a, b, (((0,), (0,)), ((), ())))
    scale = 1.0 / (dh ** 0.5)

    if wp is None:
        patches = patches_ref[0]
    else:
        hp = patches_ref[0]
        tp = hp.shape[0]
        sel = scratch[-1]

        @pl.when(pl.program_id(0) == 0)
        def _build_sel():
            yv = jax.lax.broadcasted_iota(i32, (ho, wo, tp), 0)
            xv = jax.lax.broadcasted_iota(i32, (ho, wo, tp), 1)
            tv = jax.lax.broadcasted_iota(i32, (ho, wo, tp), 2)
            base = 2 * yv * wp + 2 * xv
            for s_i, (sy, sx) in enumerate(((0, 0), (0, 1), (1, 0), (1, 1))):
                sel[s_i] = ((tv == base + (sy * wp + sx))
                            .astype(bf16).reshape(T, tp))

        hi = hp.astype(bf16)
        lo = (hp - hi.astype(f32)).astype(bf16)
        parts = []
        for s_i in range(4):
            sb = sel[s_i]
            parts.append(mm(sb, hi) + mm(sb, lo))
        patches = jnp.concatenate(parts, axis=-1)

    hb = _ln(mm(patches, wd_ref[...]) + dsb_ref[...],
             dslng_ref[...], dslnb_ref[...])

    a = _ln(hb, ln1g_ref[...], ln1b_ref[...])
    qkv = mm(a.astype(bf16), wqkv_ref[...].astype(bf16))
    q = qkv[:, :C]
    k = qkv[:, C:2 * C]
    v = qkv[:, 2 * C:]

    if topk == _NREG:
        nkp = 64
        kbd = scratch[0]
        vbd = scratch[1]

        @pl.when(pl.program_id(0) == 0)
        def _init():
            kbd[...] = jnp.zeros((nh * nkp, C), f32)
            vbd[...] = jnp.zeros((nh * nkp, C), f32)

        for h in range(nh):
            kbd[h * nkp:h * nkp + T, h * dh:(h + 1) * dh] = (
                k[:, h * dh:(h + 1) * dh])
            vbd[h * nkp:h * nkp + T, h * dh:(h + 1) * dh] = (
                v[:, h * dh:(h + 1) * dh])
        sall = mm_nt(q.astype(bf16), kbd[...].astype(bf16))
        lane = jax.lax.broadcasted_iota(i32, (T, nh * nkp), 1)
        pad = jnp.where((lane % nkp) >= T, _BIGNEG, 0.0)
        e_all = jnp.exp(sall * scale + pad)
        blk = (jax.lax.broadcasted_iota(i32, (nh * nkp, nh), 0) // nkp
               == jax.lax.broadcasted_iota(i32, (nh * nkp, nh), 1))
        blk = blk.astype(f32)
        sums = mm(e_all, blk)
        rbc = mm_nt(1.0 / sums, blk)
        p_all = (e_all * rbc).astype(bf16)
        o = mm(p_all, vbd[...].astype(bf16))
    else:
        E = _region_matrix(ho, wo, hh, ww)
        qr = mm_tn(E, q) * (1.0 / hw)
        kr = mm_tn(E, k) * (1.0 / hw)
        aff = mm_nt(qr, kr)
        mask, pick0 = _topk_mask(aff, topk)

        if topk == 1:
            kv = scratch[0]
            kvg = scratch[1]
            kv[:, :, :C] = k.reshape(ho, wo, C)
            kv[:, :, C:] = v.reshape(ho, wo, C)
            col = jax.lax.broadcasted_iota(i32, (_NREG, _NREG), 1)
            coly = jax.lax.broadcasted_iota(
                i32, (_NREG, _NWIN, _NWIN), 1).reshape(_NREG, _NREG)
            colx = jax.lax.broadcasted_iota(
                i32, (_NREG, _NWIN, _NWIN), 2).reshape(_NREG, _NREG)
            first0 = col == pick0
            oyv = jnp.sum(jnp.where(first0, coly * hh, 0),
                          axis=-1, keepdims=True)
            oxv = jnp.sum(jnp.where(first0, colx * ww, 0),
                          axis=-1, keepdims=True)
            for r in range(_NREG):
                kvg[r * hw:(r + 1) * hw, :] = (
                    kv[pl.ds(oyv[r, 0], hh), pl.ds(oxv[r, 0], ww), :]
                    .reshape(hw, 2 * C))
            kvb = kvg[...].astype(bf16)
            qb = q.astype(bf16)
            gt = _NWIN * hw
            gx = jax.lax.broadcasted_iota(i32, (hh, wo, gt), 1)
            gj = jax.lax.broadcasted_iota(i32, (hh, wo, gt), 2)
            gmask = jnp.where((gx // ww) == (gj // hw), 0.0, _BIGNEG)
            gmask = gmask.reshape(gt, gt)
            o_groups = []
            for g in range(_NWIN):
                qg = qb[g * gt:(g + 1) * gt, :]
                kvgrp = kvb[g * gt:(g + 1) * gt, :]
                o_heads = []
                for h in range(nh):
                    s = mm_nt(qg[:, h * dh:(h + 1) * dh],
                              kvgrp[:, h * dh:(h + 1) * dh])
                    e = jnp.exp(s * scale + gmask)
                    p = (e / jnp.sum(e, axis=-1, keepdims=True)).astype(bf16)
                    o_heads.append(
                        mm(p, kvgrp[:, C + h * dh:C + (h + 1) * dh]))
                o_groups.append(jnp.concatenate(o_heads, axis=-1))
            o = jnp.concatenate(o_groups, axis=0)
        else:
            breg = (1.0 - mask) * _BIGNEG
            eb = E.astype(bf16)
            ebias = mm(eb, breg.astype(bf16))
            bias = mm_nt(ebias.astype(bf16), eb)
            qb = q.astype(bf16)
            kb = k.astype(bf16)
            vb = v.astype(bf16)
            o_heads = []
            for h in range(nh):
                qh = qb[:, h * dh:(h + 1) * dh]
                kh = kb[:, h * dh:(h + 1) * dh]
                vh = vb[:, h * dh:(h + 1) * dh]
                p = _softmax(mm_nt(qh, kh) * scale + bias)
                o_heads.append(mm(p.astype(bf16), vh))
            o = jnp.concatenate(o_heads, axis=-1)

    h1 = hb + mm(o.astype(bf16), wo_ref[...].astype(bf16))
    h2 = _ln(h1, ln2g_ref[...], ln2b_ref[...])
    hid = jax.nn.gelu(mm(h2.astype(bf16), mlp1_ref[...].astype(bf16))
                      + mlp1b_ref[...])
    out = h1 + mm(hid.astype(bf16), mlp2_ref[...].astype(bf16)) + mlp2b_ref[...]
    nhwc_ref[0] = out
    nchw_ref[0] = out.T


def _run_stage(patches, weights, ho, wo, hh, ww, nh, topk, C, wp=None):
    B, tin, K = patches.shape
    T = ho * wo
    specs = [pl.BlockSpec((1, tin, K), lambda b: (b, 0, 0))]
    for w in weights:
        specs.append(pl.BlockSpec(w.shape, lambda b, n=w.ndim: (0,) * n))
    hw = hh * ww
    scratch = []
    if topk == 1:
        scratch = [pltpu.VMEM((ho, wo, 2 * C), jnp.float32),
                   pltpu.VMEM((_NREG * hw, 2 * C), jnp.float32)]
    elif topk == _NREG:
        scratch = [pltpu.VMEM((nh * 64, C), jnp.float32),
                   pltpu.VMEM((nh * 64, C), jnp.float32)]
    if wp is not None:
        scratch = scratch + [pltpu.VMEM((4, T, tin), jnp.bfloat16)]
    dimsem = ("arbitrary",) if (wp is not None or topk == _NREG) else (
        "parallel",)
    return pl.pallas_call(
        functools.partial(_stage_kernel, ho=ho, wo=wo, hh=hh, ww=ww,
                          nh=nh, topk=topk, wp=wp),
        grid=(B,),
        in_specs=specs,
        out_specs=[pl.BlockSpec((1, T, C), lambda b: (b, 0, 0)),
                   pl.BlockSpec((1, C, T), lambda b: (b, 0, 0))],
        out_shape=[jax.ShapeDtypeStruct((B, T, C), jnp.float32),
                   jax.ShapeDtypeStruct((B, C, T), jnp.float32)],
        scratch_shapes=scratch,
        compiler_params=pltpu.CompilerParams(
            dimension_semantics=dimsem,
            vmem_limit_bytes=120 * 1024 * 1024),
    )(patches, *weights)


_DIMS = [64, 128, 256, 512]
_HEADS = [2, 4, 8, 16]
_TOPK = [1, 4, 16, 49]


def kernel(x, params):
    B = x.shape[0]
    feats = []
    h_tok = None
    size = 224
    cin = 3
    for i in range(4):
        s = 4 if i == 0 else 2
        ho = size // s
        hh = ho // _NWIN
        C = _DIMS[i]
        K = s * s * cin
        if i == 0:
            patches = (x.reshape(B, cin, ho, s, ho, s)
                       .transpose(0, 2, 4, 1, 3, 5)
                       .reshape(B, ho * ho, K))
            wd = (params['dsW0'].transpose(2, 0, 1, 3).reshape(K, C))
            wprev = None
        elif i == 1:
            patches = (h_tok.reshape(B, ho, s, ho, s, cin)
                       .transpose(0, 1, 3, 2, 4, 5)
                       .reshape(B, ho * ho, K))
            wd = params['dsW%d' % i].reshape(K, C)
            wprev = None
        else:
            patches = h_tok
            wd = params['dsW%d' % i].reshape(K, C)
            wprev = size
        weights = [
            wd,
            params['dsb%d' % i].reshape(1, C),
            params['dslng%d' % i].reshape(1, C),
            params['dslnb%d' % i].reshape(1, C),
            params['ln1g%d' % i].reshape(1, C),
            params['ln1b%d' % i].reshape(1, C),
            params['wqkv%d' % i],
            params['wo%d' % i],
            params['ln2g%d' % i].reshape(1, C),
            params['ln2b%d' % i].reshape(1, C),
            params['mlp1%d' % i],
            params['mlp1b%d' % i].reshape(1, 3 * C),
            params['mlp2%d' % i],
            params['mlp2b%d' % i].reshape(1, C),
        ]
        h_nhwc, h_nchw = _run_stage(patches, weights, ho, ho, hh, hh,
                                    _HEADS[i], _TOPK[i], C, wp=wprev)
        feats.append(h_nchw.reshape(B, C, ho, ho))
        h_tok = h_nhwc
        size = ho
        cin = C
    return tuple(feats)


# speedup vs baseline: 1.0100x; 1.0100x over previous
import functools

import jax
import jax.numpy as jnp
from jax.experimental import pallas as pl
from jax.experimental.pallas import tpu as pltpu

_NWIN = 7
_NREG = _NWIN * _NWIN
_NEG = -1e30
_BIGNEG = -1e9


def _ln(x, g, b):
    mu = jnp.mean(x, axis=-1, keepdims=True)
    var = jnp.mean((x - mu) ** 2, axis=-1, keepdims=True)
    return (x - mu) * jax.lax.rsqrt(var + 1e-6) * g + b


def _softmax(s):
    e = jnp.exp(s)
    return e / jnp.sum(e, axis=-1, keepdims=True)


def _topk_mask(aff, topk):
    work = aff
    col = jax.lax.broadcasted_iota(jnp.int32, (_NREG, _NREG), 1)
    mask = jnp.zeros((_NREG, _NREG), jnp.float32)
    pick0 = None
    for t in range(topk):
        cmax = jnp.max(work, axis=-1, keepdims=True)
        pick = jnp.min(jnp.where(work >= cmax, col, _NREG),
                       axis=-1, keepdims=True)
        if t == 0:
            pick0 = pick
        first = col == pick
        mask = mask + first.astype(jnp.float32)
        work = jnp.where(first, _NEG, work)
    return mask, pick0


def _region_matrix(ho, wo, hh, ww):
    y = jax.lax.broadcasted_iota(jnp.int32, (ho, wo, _NREG), 0)
    x = jax.lax.broadcasted_iota(jnp.int32, (ho, wo, _NREG), 1)
    r = jax.lax.broadcasted_iota(jnp.int32, (ho, wo, _NREG), 2)
    e3 = ((y // hh) * _NWIN + (x // ww)) == r
    return e3.astype(jnp.float32).reshape(ho * wo, _NREG)


def _stage_kernel(patches_ref, wd_ref, dsb_ref, dslng_ref, dslnb_ref,
                  ln1g_ref, ln1b_ref, wqkv_ref, wo_ref,
                  ln2g_ref, ln2b_ref, mlp1_ref, mlp1b_ref,
                  mlp2_ref, mlp2b_ref, nhwc_ref, nchw_ref, *scratch,
                  ho, wo, hh, ww, nh, topk, wp):
    C = wo_ref.shape[0]
    dh = C // nh
    hw = hh * ww
    T = _NREG * hw
    f32 = jnp.float32
    bf16 = jnp.bfloat16
    i32 = jnp.int32
    dot = functools.partial(jax.lax.dot_general, preferred_element_type=f32)
    mm = lambda a, b: dot(a, b, (((1,), (0,)), ((), ())))
    mm_nt = lambda a, b: dot(a, b, (((1,), (1,)), ((), ())))
    mm_tn = lambda a, b: dot(a, b, (((0,), (0,)), ((), ())))
    scale = 1.0 / (dh ** 0.5)

    if wp is None:
        patches = patches_ref[0]
    else:
        hp = patches_ref[0]
        tp = hp.shape[0]
        sel = scratch[-1]

        @pl.when(pl.program_id(0) == 0)
        def _build_sel():
            yv = jax.lax.broadcasted_iota(i32, (ho, wo, tp), 0)
            xv = jax.lax.broadcasted_iota(i32, (ho, wo, tp), 1)
            tv = jax.lax.broadcasted_iota(i32, (ho, wo, tp), 2)
            base = 2 * yv * wp + 2 * xv
            for s_i, (sy, sx) in enumerate(((0, 0), (0, 1), (1, 0), (1, 1))):
                sel[s_i] = ((tv == base + (sy * wp + sx))
                            .astype(bf16).reshape(T, tp))

        hi = hp.astype(bf16)
        lo = (hp - hi.astype(f32)).astype(bf16)
        parts = []
        for s_i in range(4):
            sb = sel[s_i]
            parts.append(mm(sb, hi) + mm(sb, lo))
        patches = jnp.concatenate(parts, axis=-1)

    hb = _ln(mm(patches, wd_ref[...]) + dsb_ref[...],
             dslng_ref[...], dslnb_ref[...])

    a = _ln(hb, ln1g_ref[...], ln1b_ref[...])
    qkv = mm(a.astype(bf16), wqkv_ref[...].astype(bf16))
    q = qkv[:, :C]
    k = qkv[:, C:2 * C]
    v = qkv[:, 2 * C:]

    if topk == _NREG:
        nkp = 64
        kbd = scratch[0]
        vbd = scratch[1]

        @pl.when(pl.program_id(0) == 0)
        def _init():
            kbd[...] = jnp.zeros((nh * nkp, C), f32)
            vbd[...] = jnp.zeros((nh * nkp, C), f32)

        for h in range(nh):
            kbd[h * nkp:h * nkp + T, h * dh:(h + 1) * dh] = (
                k[:, h * dh:(h + 1) * dh])
            vbd[h * nkp:h * nkp + T, h * dh:(h + 1) * dh] = (
                v[:, h * dh:(h + 1) * dh])
        sall = mm_nt(q.astype(bf16), kbd[...].astype(bf16))
        lane = jax.lax.broadcasted_iota(i32, (T, nh * nkp), 1)
        pad = jnp.where((lane % nkp) >= T, _BIGNEG, 0.0)
        e_all = jnp.exp(sall * scale + pad)
        blk = (jax.lax.broadcasted_iota(i32, (nh * nkp, nh), 0) // nkp
               == jax.lax.broadcasted_iota(i32, (nh * nkp, nh), 1))
        blk = blk.astype(f32)
        sums = mm(e_all, blk)
        rbc = mm_nt(1.0 / sums, blk)
        p_all = (e_all * rbc).astype(bf16)
        o = mm(p_all, vbd[...].astype(bf16))
    else:
        E = _region_matrix(ho, wo, hh, ww)
        qr = mm_tn(E, q) * (1.0 / hw)
        kr = mm_tn(E, k) * (1.0 / hw)
        aff = mm_nt(qr, kr)
        mask, pick0 = _topk_mask(aff, topk)

        if topk == 1:
            kv = scratch[0]
            kvg = scratch[1]
            kv[:, :, :C] = k.reshape(ho, wo, C)
            kv[:, :, C:] = v.reshape(ho, wo, C)
            col = jax.lax.broadcasted_iota(i32, (_NREG, _NREG), 1)
            coly = jax.lax.broadcasted_iota(
                i32, (_NREG, _NWIN, _NWIN), 1).reshape(_NREG, _NREG)
            colx = jax.lax.broadcasted_iota(
                i32, (_NREG, _NWIN, _NWIN), 2).reshape(_NREG, _NREG)
            first0 = col == pick0
            oyv = jnp.sum(jnp.where(first0, coly * hh, 0),
                          axis=-1, keepdims=True)
            oxv = jnp.sum(jnp.where(first0, colx * ww, 0),
                          axis=-1, keepdims=True)
            for r in range(_NREG):
                kvg[r * hw:(r + 1) * hw, :] = (
                    kv[pl.ds(oyv[r, 0], hh), pl.ds(oxv[r, 0], ww), :]
                    .reshape(hw, 2 * C))
            kvb = kvg[...].astype(bf16)
            qb = q.astype(bf16)
            gt = _NWIN * hw
            gx = jax.lax.broadcasted_iota(i32, (hh, wo, gt), 1)
            gj = jax.lax.broadcasted_iota(i32, (hh, wo, gt), 2)
            gmask = jnp.where((gx // ww) == (gj // hw), 0.0, _BIGNEG)
            gmask = gmask.reshape(gt, gt)
            o_groups = []
            for g in range(_NWIN):
                qg = qb[g * gt:(g + 1) * gt, :]
                kvgrp = kvb[g * gt:(g + 1) * gt, :]
                o_heads = []
                for h in range(nh):
                    s = mm_nt(qg[:, h * dh:(h + 1) * dh],
                              kvgrp[:, h * dh:(h + 1) * dh])
                    e = jnp.exp(s * scale + gmask)
                    p = (e / jnp.sum(e, axis=-1, keepdims=True)).astype(bf16)
                    o_heads.append(
                        mm(p, kvgrp[:, C + h * dh:C + (h + 1) * dh]))
                o_groups.append(jnp.concatenate(o_heads, axis=-1))
            o = jnp.concatenate(o_groups, axis=0)
        else:
            breg = (1.0 - mask) * _BIGNEG
            eb = E.astype(bf16)
            ebias = mm(eb, breg.astype(bf16))
            bias = mm_nt(ebias.astype(bf16), eb)
            qb = q.astype(bf16)
            kb = k.astype(bf16)
            vb = v.astype(bf16)
            o_heads = []
            for h in range(nh):
                qh = qb[:, h * dh:(h + 1) * dh]
                kh = kb[:, h * dh:(h + 1) * dh]
                vh = vb[:, h * dh:(h + 1) * dh]
                p = _softmax(mm_nt(qh, kh) * scale + bias)
                o_heads.append(mm(p.astype(bf16), vh))
            o = jnp.concatenate(o_heads, axis=-1)

    h1 = hb + mm(o.astype(bf16), wo_ref[...].astype(bf16))
    h2 = _ln(h1, ln2g_ref[...], ln2b_ref[...])
    hid = jax.nn.gelu(mm(h2.astype(bf16), mlp1_ref[...].astype(bf16))
                      + mlp1b_ref[...])
    out = h1 + mm(hid.astype(bf16), mlp2_ref[...].astype(bf16)) + mlp2b_ref[...]
    nhwc_ref[0] = out
    nchw_ref[0] = out.T


def _run_stage(patches, weights, ho, wo, hh, ww, nh, topk, C, wp=None):
    B, tin, K = patches.shape
    T = ho * wo
    specs = [pl.BlockSpec((1, tin, K), lambda b: (b, 0, 0))]
    for w in weights:
        specs.append(pl.BlockSpec(w.shape, lambda b, n=w.ndim: (0,) * n))
    hw = hh * ww
    scratch = []
    if topk == 1:
        scratch = [pltpu.VMEM((ho, wo, 2 * C), jnp.float32),
                   pltpu.VMEM((_NREG * hw, 2 * C), jnp.float32)]
    elif topk == _NREG:
        scratch = [pltpu.VMEM((nh * 64, C), jnp.float32),
                   pltpu.VMEM((nh * 64, C), jnp.float32)]
    if wp is not None:
        scratch = scratch + [pltpu.VMEM((4, T, tin), jnp.bfloat16)]
    dimsem = ("arbitrary",) if (wp is not None or topk == _NREG) else (
        "parallel",)
    return pl.pallas_call(
        functools.partial(_stage_kernel, ho=ho, wo=wo, hh=hh, ww=ww,
                          nh=nh, topk=topk, wp=wp),
        grid=(B,),
        in_specs=specs,
        out_specs=[pl.BlockSpec((1, T, C), lambda b: (b, 0, 0)),
                   pl.BlockSpec((1, C, T), lambda b: (b, 0, 0))],
        out_shape=[jax.ShapeDtypeStruct((B, T, C), jnp.float32),
                   jax.ShapeDtypeStruct((B, C, T), jnp.float32)],
        scratch_shapes=scratch,
        compiler_params=pltpu.CompilerParams(
            dimension_semantics=dimsem,
            vmem_limit_bytes=120 * 1024 * 1024),
    )(patches, *weights)


_DIMS = [64, 128, 256, 512]
_HEADS = [2, 4, 8, 16]
_TOPK = [1, 4, 16, 49]


def kernel(x, params):
    B = x.shape[0]
    feats = []
    h_tok = None
    size = 224
    cin = 3
    for i in range(4):
        s = 4 if i == 0 else 2
        ho = size // s
        hh = ho // _NWIN
        C = _DIMS[i]
        K = s * s * cin
        if i == 0:
            patches = (x.reshape(B, cin, ho, s, ho, s)
                       .transpose(0, 2, 4, 1, 3, 5)
                       .reshape(B, ho * ho, K))
            wd = (params['dsW0'].transpose(2, 0, 1, 3).reshape(K, C))
            wprev = None
        elif i == 1:
            h6 = h_tok.reshape(B, ho, s, ho, s, cin)
            patches = jnp.concatenate(
                [h6[:, :, sy, :, sx, :] for sy in range(s)
                 for sx in range(s)],
                axis=-1).reshape(B, ho * ho, K)
            wd = params['dsW%d' % i].reshape(K, C)
            wprev = None
        else:
            patches = h_tok
            wd = params['dsW%d' % i].reshape(K, C)
            wprev = size
        weights = [
            wd,
            params['dsb%d' % i].reshape(1, C),
            params['dslng%d' % i].reshape(1, C),
            params['dslnb%d' % i].reshape(1, C),
            params['ln1g%d' % i].reshape(1, C),
            params['ln1b%d' % i].reshape(1, C),
            params['wqkv%d' % i],
            params['wo%d' % i],
            params['ln2g%d' % i].reshape(1, C),
            params['ln2b%d' % i].reshape(1, C),
            params['mlp1%d' % i],
            params['mlp1b%d' % i].reshape(1, 3 * C),
            params['mlp2%d' % i],
            params['mlp2b%d' % i].reshape(1, C),
        ]
        h_nhwc, h_nchw = _run_stage(patches, weights, ho, ho, hh, hh,
                                    _HEADS[i], _TOPK[i], C, wp=wprev)
        feats.append(h_nchw.reshape(B, C, ho, ho))
        h_tok = h_nhwc
        size = ho
        cin = C
    return tuple(feats)


# speedup vs baseline: 1.0122x; 1.0022x over previous
import functools

import jax
import jax.numpy as jnp
from jax.experimental import pallas as pl
from jax.experimental.pallas import tpu as pltpu

_NWIN = 7
_NREG = _NWIN * _NWIN
_NEG = -1e30
_BIGNEG = -1e9


def _ln(x, g, b):
    mu = jnp.mean(x, axis=-1, keepdims=True)
    var = jnp.mean((x - mu) ** 2, axis=-1, keepdims=True)
    return (x - mu) * jax.lax.rsqrt(var + 1e-6) * g + b


def _softmax(s):
    e = jnp.exp(s)
    return e / jnp.sum(e, axis=-1, keepdims=True)


def _topk_mask(aff, topk):
    work = aff
    col = jax.lax.broadcasted_iota(jnp.int32, (_NREG, _NREG), 1)
    mask = jnp.zeros((_NREG, _NREG), jnp.float32)
    pick0 = None
    for t in range(topk):
        cmax = jnp.max(work, axis=-1, keepdims=True)
        pick = jnp.min(jnp.where(work >= cmax, col, _NREG),
                       axis=-1, keepdims=True)
        if t == 0:
            pick0 = pick
        first = col == pick
        mask = mask + first.astype(jnp.float32)
        work = jnp.where(first, _NEG, work)
    return mask, pick0


def _region_matrix(ho, wo, hh, ww):
    y = jax.lax.broadcasted_iota(jnp.int32, (ho, wo, _NREG), 0)
    x = jax.lax.broadcasted_iota(jnp.int32, (ho, wo, _NREG), 1)
    r = jax.lax.broadcasted_iota(jnp.int32, (ho, wo, _NREG), 2)
    e3 = ((y // hh) * _NWIN + (x // ww)) == r
    return e3.astype(jnp.float32).reshape(ho * wo, _NREG)


def _stage_kernel(patches_ref, wd_ref, dsb_ref, dslng_ref, dslnb_ref,
                  ln1g_ref, ln1b_ref, wqkv_ref, wo_ref,
                  ln2g_ref, ln2b_ref, mlp1_ref, mlp1b_ref,
                  mlp2_ref, mlp2b_ref, nhwc_ref, nchw_ref, *scratch,
                  ho, wo, hh, ww, nh, topk, wp):
    C = wo_ref.shape[0]
    dh = C // nh
    hw = hh * ww
    T = _NREG * hw
    f32 = jnp.float32
    bf16 = jnp.bfloat16
    i32 = jnp.int32
    dot = functools.partial(jax.lax.dot_general, preferred_element_type=f32)
    mm = lambda a, b: dot(a, b, (((1,), (0,)), ((), ())))
    mm_nt = lambda a, b: dot(a, b, (((1,), (1,)), ((), ())))
    mm_tn = lambda a, b: dot(a, b, (((0,), (0,)), ((), ())))
    scale = 1.0 / (dh ** 0.5)

    if wp is None:
        patches = patches_ref[0]
    else:
        hp = patches_ref[0]
        tp = hp.shape[0]
        sel = scratch[-1]

        @pl.when(pl.program_id(0) == 0)
        def _build_sel():
            yv = jax.lax.broadcasted_iota(i32, (ho, wo, tp), 0)
            xv = jax.lax.broadcasted_iota(i32, (ho, wo, tp), 1)
            tv = jax.lax.broadcasted_iota(i32, (ho, wo, tp), 2)
            base = 2 * yv * wp + 2 * xv
            for s_i, (sy, sx) in enumerate(((0, 0), (0, 1), (1, 0), (1, 1))):
                sel[s_i] = ((tv == base + (sy * wp + sx))
                            .astype(bf16).reshape(T, tp))

        hi = hp.astype(bf16)
        lo = (hp - hi.astype(f32)).astype(bf16)
        parts = []
        for s_i in range(4):
            sb = sel[s_i]
            parts.append(mm(sb, hi) + mm(sb, lo))
        patches = jnp.concatenate(parts, axis=-1)

    hb = _ln(mm(patches, wd_ref[...]) + dsb_ref[...],
             dslng_ref[...], dslnb_ref[...])

    a = _ln(hb, ln1g_ref[...], ln1b_ref[...])
    qkv = mm(a.astype(bf16), wqkv_ref[...].astype(bf16))
    q = qkv[:, :C]
    k = qkv[:, C:2 * C]
    v = qkv[:, 2 * C:]

    if topk == _NREG:
        nkp = 64
        kbd = scratch[0]
        vbd = scratch[1]

        @pl.when(pl.program_id(0) == 0)
        def _init():
            kbd[...] = jnp.zeros((nh * nkp, C), f32)
            vbd[...] = jnp.zeros((nh * nkp, C), f32)

        for h in range(nh):
            kbd[h * nkp:h * nkp + T, h * dh:(h + 1) * dh] = (
                k[:, h * dh:(h + 1) * dh])
            vbd[h * nkp:h * nkp + T, h * dh:(h + 1) * dh] = (
                v[:, h * dh:(h + 1) * dh])
        sall = mm_nt((q * scale).astype(bf16),
                     kbd[...].astype(bf16))
        lane = jax.lax.broadcasted_iota(i32, (T, nh * nkp), 1)
        pad = jnp.where((lane % nkp) >= T, _BIGNEG, 0.0)
        e_all = jnp.exp(sall + pad)
        blk = (jax.lax.broadcasted_iota(i32, (nh * nkp, nh), 0) // nkp
               == jax.lax.broadcasted_iota(i32, (nh * nkp, nh), 1))
        blk = blk.astype(f32)
        sums = mm(e_all, blk)
        rbc = mm_nt(1.0 / sums, blk)
        p_all = (e_all * rbc).astype(bf16)
        o = mm(p_all, vbd[...].astype(bf16))
    else:
        E = _region_matrix(ho, wo, hh, ww)
        qr = mm_tn(E, q) * (1.0 / hw)
        kr = mm_tn(E, k) * (1.0 / hw)
        aff = mm_nt(qr, kr)
        mask, pick0 = _topk_mask(aff, topk)

        if topk == 1:
            kv = scratch[0]
            kvg = scratch[1]
            kv[:, :, :C] = k.reshape(ho, wo, C)
            kv[:, :, C:] = v.reshape(ho, wo, C)
            col = jax.lax.broadcasted_iota(i32, (_NREG, _NREG), 1)
            coly = jax.lax.broadcasted_iota(
                i32, (_NREG, _NWIN, _NWIN), 1).reshape(_NREG, _NREG)
            colx = jax.lax.broadcasted_iota(
                i32, (_NREG, _NWIN, _NWIN), 2).reshape(_NREG, _NREG)
            first0 = col == pick0
            oyv = jnp.sum(jnp.where(first0, coly * hh, 0),
                          axis=-1, keepdims=True)
            oxv = jnp.sum(jnp.where(first0, colx * ww, 0),
                          axis=-1, keepdims=True)
            for r in range(_NREG):
                kvg[r * hw:(r + 1) * hw, :] = (
                    kv[pl.ds(oyv[r, 0], hh), pl.ds(oxv[r, 0], ww), :]
                    .reshape(hw, 2 * C))
            kvb = kvg[...].astype(bf16)
            qb = (q * scale).astype(bf16)
            gt = _NWIN * hw
            gx = jax.lax.broadcasted_iota(i32, (hh, wo, gt), 1)
            gj = jax.lax.broadcasted_iota(i32, (hh, wo, gt), 2)
            gmask = jnp.where((gx // ww) == (gj // hw), 0.0, _BIGNEG)
            gmask = gmask.reshape(gt, gt)
            o_groups = []
            for g in range(_NWIN):
                qg = qb[g * gt:(g + 1) * gt, :]
                kvgrp = kvb[g * gt:(g + 1) * gt, :]
                o_heads = []
                for h in range(nh):
                    s = mm_nt(qg[:, h * dh:(h + 1) * dh],
                              kvgrp[:, h * dh:(h + 1) * dh])
                    e = jnp.exp(s + gmask)
                    p = (e / jnp.sum(e, axis=-1, keepdims=True)).astype(bf16)
                    o_heads.append(
                        mm(p, kvgrp[:, C + h * dh:C + (h + 1) * dh]))
                o_groups.append(jnp.concatenate(o_heads, axis=-1))
            o = jnp.concatenate(o_groups, axis=0)
        else:
            breg = (1.0 - mask) * _BIGNEG
            eb = E.astype(bf16)
            ebias = mm(eb, breg.astype(bf16))
            bias = mm_nt(ebias.astype(bf16), eb)
            qb = (q * scale).astype(bf16)
            kb = k.astype(bf16)
            vb = v.astype(bf16)
            o_heads = []
            for h in range(nh):
                qh = qb[:, h * dh:(h + 1) * dh]
                kh = kb[:, h * dh:(h + 1) * dh]
                vh = vb[:, h * dh:(h + 1) * dh]
                p = _softmax(mm_nt(qh, kh) + bias)
                o_heads.append(mm(p.astype(bf16), vh))
            o = jnp.concatenate(o_heads, axis=-1)

    h1 = hb + mm(o.astype(bf16), wo_ref[...].astype(bf16))
    h2 = _ln(h1, ln2g_ref[...], ln2b_ref[...])
    hid = jax.nn.gelu(mm(h2.astype(bf16), mlp1_ref[...].astype(bf16))
                      + mlp1b_ref[...])
    out = h1 + mm(hid.astype(bf16), mlp2_ref[...].astype(bf16)) + mlp2b_ref[...]
    nhwc_ref[0] = out
    nchw_ref[0] = out.T


def _run_stage(patches, weights, ho, wo, hh, ww, nh, topk, C, wp=None):
    B, tin, K = patches.shape
    T = ho * wo
    specs = [pl.BlockSpec((1, tin, K), lambda b: (b, 0, 0))]
    for w in weights:
        specs.append(pl.BlockSpec(w.shape, lambda b, n=w.ndim: (0,) * n))
    hw = hh * ww
    scratch = []
    if topk == 1:
        scratch = [pltpu.VMEM((ho, wo, 2 * C), jnp.float32),
                   pltpu.VMEM((_NREG * hw, 2 * C), jnp.float32)]
    elif topk == _NREG:
        scratch = [pltpu.VMEM((nh * 64, C), jnp.float32),
                   pltpu.VMEM((nh * 64, C), jnp.float32)]
    if wp is not None:
        scratch = scratch + [pltpu.VMEM((4, T, tin), jnp.bfloat16)]
    dimsem = ("arbitrary",) if (wp is not None or topk == _NREG) else (
        "parallel",)
    return pl.pallas_call(
        functools.partial(_stage_kernel, ho=ho, wo=wo, hh=hh, ww=ww,
                          nh=nh, topk=topk, wp=wp),
        grid=(B,),
        in_specs=specs,
        out_specs=[pl.BlockSpec((1, T, C), lambda b: (b, 0, 0)),
                   pl.BlockSpec((1, C, T), lambda b: (b, 0, 0))],
        out_shape=[jax.ShapeDtypeStruct((B, T, C), jnp.float32),
                   jax.ShapeDtypeStruct((B, C, T), jnp.float32)],
        scratch_shapes=scratch,
        compiler_params=pltpu.CompilerParams(
            dimension_semantics=dimsem,
            vmem_limit_bytes=120 * 1024 * 1024),
    )(patches, *weights)


_DIMS = [64, 128, 256, 512]
_HEADS = [2, 4, 8, 16]
_TOPK = [1, 4, 16, 49]


def kernel(x, params):
    B = x.shape[0]
    feats = []
    h_tok = None
    size = 224
    cin = 3
    for i in range(4):
        s = 4 if i == 0 else 2
        ho = size // s
        hh = ho // _NWIN
        C = _DIMS[i]
        K = s * s * cin
        if i == 0:
            patches = (x.reshape(B, cin, ho, s, ho, s)
                       .transpose(0, 2, 4, 1, 3, 5)
                       .reshape(B, ho * ho, K))
            wd = (params['dsW0'].transpose(2, 0, 1, 3).reshape(K, C))
            wprev = None
        elif i == 1:
            h6 = h_tok.reshape(B, ho, s, ho, s, cin)
            patches = jnp.concatenate(
                [h6[:, :, sy, :, sx, :] for sy in range(s)
                 for sx in range(s)],
                axis=-1).reshape(B, ho * ho, K)
            wd = params['dsW%d' % i].reshape(K, C)
            wprev = None
        else:
            patches = h_tok
            wd = params['dsW%d' % i].reshape(K, C)
            wprev = size
        weights = [
            wd,
            params['dsb%d' % i].reshape(1, C),
            params['dslng%d' % i].reshape(1, C),
            params['dslnb%d' % i].reshape(1, C),
            params['ln1g%d' % i].reshape(1, C),
            params['ln1b%d' % i].reshape(1, C),
            params['wqkv%d' % i],
            params['wo%d' % i],
            params['ln2g%d' % i].reshape(1, C),
            params['ln2b%d' % i].reshape(1, C),
            params['mlp1%d' % i],
            params['mlp1b%d' % i].reshape(1, 3 * C),
            params['mlp2%d' % i],
            params['mlp2b%d' % i].reshape(1, C),
        ]
        h_nhwc, h_nchw = _run_stage(patches, weights, ho, ho, hh, hh,
                                    _HEADS[i], _TOPK[i], C, wp=wprev)
        feats.append(h_nchw.reshape(B, C, ho, ho))
        h_tok = h_nhwc
        size = ho
        cin = C
    return tuple(feats)
